# Initial kernel scaffold; baseline (speedup 1.0000x reference)
import jax
import jax.numpy as jnp
from jax.experimental import pallas as pl


def kernel(Zhat, ab_gt):
    raise NotImplementedError("write your pallas kernel here")


# speedup vs baseline: 10.0006x; 10.0006x over previous
import functools

import jax
import jax.numpy as jnp
from jax import lax
from jax.experimental import pallas as pl

_Q = 529
_HW = 56 * 56
_PT = _HW


def _loss_body(z_ref, a_ref, b_ref, out_ref):
    bi = pl.program_id(0)

    x = z_ref[0]
    m = jnp.max(x, axis=0, keepdims=True)
    s = jnp.sum(jnp.exp(x - m), axis=0, keepdims=True)
    lse = m + jnp.log(s)

    a = a_ref[0]
    b = b_ref[0]

    qi = lax.broadcasted_iota(jnp.int32, (_Q, _PT), 0)
    ca = ((qi // 23) * 10 - 110).astype(jnp.float32)
    cb = ((qi % 23) * 10 - 110).astype(jnp.float32)
    d2 = (a - ca) ** 2 + (b - cb) ** 2

    acc_w = jnp.zeros((1, _PT), jnp.float32)
    acc_wz = jnp.zeros((1, _PT), jnp.float32)
    big = jnp.float32(3.4e38)
    for _ in range(5):
        mmin = jnp.min(d2, axis=0, keepdims=True)
        idx = jnp.min(jnp.where(d2 == mmin, qi, _Q), axis=0, keepdims=True)
        sel = qi == idx
        w = jnp.exp(-mmin / 50.0)
        acc_w += w
        acc_wz += w * jnp.sum(jnp.where(sel, x, 0.0), axis=0, keepdims=True)
        d2 = jnp.where(sel, big, d2)

    part = jnp.sum(lse - acc_wz / acc_w, axis=1, keepdims=True)

    @pl.when(bi == 0)
    def _():
        out_ref[...] = jnp.zeros((1, 1), jnp.float32)

    out_ref[...] += part


@jax.jit
def kernel(Zhat, ab_gt):
    B, Q, H, W = Zhat.shape
    ab_ds = jax.image.resize(ab_gt, (B, 2, H, W), method="bilinear",
                             antialias=False)
    a_img = ab_ds[:, 0].reshape(B, 1, H * W)
    b_img = ab_ds[:, 1].reshape(B, 1, H * W)
    z = Zhat.reshape(B, Q, H * W)

    total = pl.pallas_call(
        _loss_body,
        grid=(B,),
        in_specs=[
            pl.BlockSpec((1, Q, _PT), lambda bi: (bi, 0, 0)),
            pl.BlockSpec((1, 1, _PT), lambda bi: (bi, 0, 0)),
            pl.BlockSpec((1, 1, _PT), lambda bi: (bi, 0, 0)),
        ],
        out_specs=pl.BlockSpec((1, 1), lambda bi: (0, 0)),
        out_shape=jax.ShapeDtypeStruct((1, 1), jnp.float32),
    )(z, a_img, b_img)

    return total[0, 0] / jnp.float32(B * H * W)


# speedup vs baseline: 11.5489x; 1.1548x over previous
import functools

import jax
import jax.numpy as jnp
from jax import lax
from jax.experimental import pallas as pl

_Q = 529
_HW = 56 * 56
_PT = _HW


def _loss_body(z_ref, a_ref, b_ref, ca_ref, cb_ref, out_ref):
    bi = pl.program_id(0)

    x = z_ref[0]
    m = jnp.max(x, axis=0, keepdims=True)
    s = jnp.sum(jnp.exp(x - m), axis=0, keepdims=True)
    lse = m + jnp.log(s)

    a = a_ref[0]
    b = b_ref[0]

    ca = ca_ref[...]
    cb = cb_ref[...]
    d2 = (a - ca) ** 2 + (b - cb) ** 2

    acc_w = jnp.zeros((1, _PT), jnp.float32)
    acc_wz = jnp.zeros((1, _PT), jnp.float32)
    big = jnp.float32(3.4e38)
    for _ in range(5):
        mmin = jnp.min(d2, axis=0, keepdims=True)
        sel = d2 == mmin
        w = jnp.exp(-mmin / 50.0)
        acc_w += w
        acc_wz += w * jnp.sum(jnp.where(sel, x, 0.0), axis=0, keepdims=True)
        d2 = jnp.where(sel, big, d2)

    part = jnp.sum(lse - acc_wz / acc_w, axis=1, keepdims=True)

    @pl.when(bi == 0)
    def _():
        out_ref[...] = jnp.zeros((1, 1), jnp.float32)

    out_ref[...] += part


@jax.jit
def kernel(Zhat, ab_gt):
    B, Q, H, W = Zhat.shape
    ab_ds = jax.image.resize(ab_gt, (B, 2, H, W), method="bilinear",
                             antialias=False)
    a_img = ab_ds[:, 0].reshape(B, 1, H * W)
    b_img = ab_ds[:, 1].reshape(B, 1, H * W)
    z = Zhat.reshape(B, Q, H * W)
    qs = jnp.arange(Q, dtype=jnp.int32)
    ca = ((qs // 23) * 10 - 110).astype(jnp.float32).reshape(Q, 1)
    cb = ((qs % 23) * 10 - 110).astype(jnp.float32).reshape(Q, 1)

    total = pl.pallas_call(
        _loss_body,
        grid=(B,),
        in_specs=[
            pl.BlockSpec((1, Q, _PT), lambda bi: (bi, 0, 0)),
            pl.BlockSpec((1, 1, _PT), lambda bi: (bi, 0, 0)),
            pl.BlockSpec((1, 1, _PT), lambda bi: (bi, 0, 0)),
            pl.BlockSpec((Q, 1), lambda bi: (0, 0)),
            pl.BlockSpec((Q, 1), lambda bi: (0, 0)),
        ],
        out_specs=pl.BlockSpec((1, 1), lambda bi: (0, 0)),
        out_shape=jax.ShapeDtypeStruct((1, 1), jnp.float32),
    )(z, a_img, b_img, ca, cb)

    return total[0, 0] / jnp.float32(B * H * W)
